# baseline (speedup 1.0000x reference)
import functools

import jax
import jax.numpy as jnp
from jax import lax
from jax.experimental import pallas as pl
from jax.experimental.pallas import tpu as pltpu
from jax.experimental.pallas import tpu_sc as plsc


@functools.lru_cache(maxsize=None)
def _make_gather(V, D, B):
    info = plsc.get_sparse_core_info()
    NC, NS = info.num_cores, info.num_subcores
    NW = NC * NS
    assert B % (8 * NW) == 0
    b_per_w = B // NW
    mesh = plsc.VectorSubcoreMesh(core_axis_name="c", subcore_axis_name="s")

    @functools.partial(
        pl.kernel,
        mesh=mesh,
        out_type=jax.ShapeDtypeStruct((B, 2 * D), jnp.float32),
        scratch_types=[
            pltpu.VMEM((b_per_w,), jnp.int32),
            pltpu.VMEM((b_per_w, 2 * D), jnp.float32),
            pltpu.SemaphoreType.DMA,
        ],
    )
    def k(tbl_hbm, idx_hbm, out_hbm, lab_v, rows_v, sem):
        wid = lax.axis_index("s") * NC + lax.axis_index("c")
        base = wid * b_per_w
        pltpu.sync_copy(idx_hbm.at[pl.ds(base, b_per_w)], lab_v)
        pltpu.async_copy(tbl_hbm.at[lab_v], rows_v, sem).wait()
        pltpu.sync_copy(rows_v, out_hbm.at[pl.ds(base, b_per_w)])

    return k


def kernel(labels, table):
    B, = labels.shape
    V, D = table.shape
    tp = jnp.concatenate([table, table], axis=1)
    out128 = _make_gather(V, D, B)(tp, labels.astype(jnp.int32))
    return out128[:, :D]


# speedup vs baseline: 1.6445x; 1.6445x over previous
import functools

import jax
import jax.numpy as jnp
from jax import lax
from jax.experimental import pallas as pl
from jax.experimental.pallas import tpu as pltpu
from jax.experimental.pallas import tpu_sc as plsc


@functools.lru_cache(maxsize=None)
def _make_gather(V, D, B):
    info = plsc.get_sparse_core_info()
    NC, NS, L = info.num_cores, info.num_subcores, info.num_lanes
    NW = NC * NS
    assert B % (8 * NW) == 0 and D == 64
    b_per_w = B // NW
    chunk = 32
    n_chunks = b_per_w // chunk
    groups_per_chunk = chunk // L
    mesh = plsc.VectorSubcoreMesh(core_axis_name="c", subcore_axis_name="s")

    @functools.partial(
        pl.kernel,
        mesh=mesh,
        compiler_params=pltpu.CompilerParams(needs_layout_passes=False),
        out_type=jax.ShapeDtypeStruct((B, 2 * D), jnp.float32),
        scratch_types=[
            pltpu.VMEM((b_per_w,), jnp.int32),
            pltpu.VMEM((chunk, 8, D), jnp.float32),
            pltpu.VMEM((b_per_w, 2 * D), jnp.float32),
            pltpu.SemaphoreType.DMA,
        ],
    )
    def k(tbl_hbm, idx_hbm, out_hbm, lab_v, slots_v, outb_v, sem):
        wid = lax.axis_index("s") * NC + lax.axis_index("c")
        base = wid * b_per_w
        pltpu.sync_copy(idx_hbm.at[pl.ds(base, b_per_w)], lab_v)

        def per_chunk(c, carry):
            for g in range(groups_per_chunk):
                lab16 = lab_v[pl.ds(c * chunk + g * L, L)]
                for kk in range(L):
                    lb = pl.multiple_of((lab16[kk] >> 3) * 8, 8)
                    pltpu.async_copy(
                        tbl_hbm.at[pl.ds(lb, 8), :],
                        slots_v.at[g * L + kk],
                        sem,
                    )
            for _ in range(chunk):
                pltpu.make_async_copy(
                    tbl_hbm.at[pl.ds(0, 8), :], slots_v.at[0], sem
                ).wait()
            for g in range(groups_per_chunk):
                off = c * chunk + g * L
                rloc = lax.iota(jnp.int32, L) + g * L
                rglob = rloc + c * chunk
                h = lab_v[pl.ds(off, L)] & 7
                for d in range(D):
                    dsplat = jnp.full((L,), d, jnp.int32)
                    vals = plsc.load_gather(slots_v, [rloc, h, dsplat])
                    plsc.store_scatter(outb_v, [rglob, dsplat], vals)
            return carry

        lax.fori_loop(0, n_chunks, per_chunk, 0)
        pltpu.sync_copy(outb_v, out_hbm.at[pl.ds(base, b_per_w)])

    return k


def kernel(labels, table):
    B, = labels.shape
    V, D = table.shape
    out128 = _make_gather(V, D, B)(table, labels.astype(jnp.int32))
    return out128[:, :D]


# speedup vs baseline: 1.7721x; 1.0776x over previous
import functools

import jax
import jax.numpy as jnp
from jax import lax
from jax.experimental import pallas as pl
from jax.experimental.pallas import tpu as pltpu
from jax.experimental.pallas import tpu_sc as plsc


@functools.lru_cache(maxsize=None)
def _make_gather(V, D, B):
    info = plsc.get_sparse_core_info()
    NC, NS, L = info.num_cores, info.num_subcores, info.num_lanes
    NW = NC * NS
    assert B % (8 * NW) == 0 and D == 64
    b_per_w = B // NW
    chunk = 32
    n_chunks = b_per_w // chunk
    groups_per_chunk = chunk // L
    mesh = plsc.VectorSubcoreMesh(core_axis_name="c", subcore_axis_name="s")

    @functools.partial(
        pl.kernel,
        mesh=mesh,
        out_type=jax.ShapeDtypeStruct((B, 2 * D), jnp.float32),
        scratch_types=[
            pltpu.VMEM((b_per_w,), jnp.int32),
            pltpu.VMEM((chunk, 8, D), jnp.float32),
            pltpu.VMEM((b_per_w, 2 * D), jnp.float32),
            pltpu.SemaphoreType.DMA,
        ],
    )
    def k(tbl_hbm, idx_hbm, out_hbm, lab_v, slots_v, outb_v, sem):
        wid = lax.axis_index("s") * NC + lax.axis_index("c")
        base = wid * b_per_w
        pltpu.sync_copy(idx_hbm.at[pl.ds(base, b_per_w)], lab_v)

        def per_chunk(c, carry):
            for g in range(groups_per_chunk):
                lab16 = lab_v[pl.ds(c * chunk + g * L, L)]
                for kk in range(L):
                    lb = pl.multiple_of((lab16[kk] >> 3) * 8, 8)
                    pltpu.async_copy(
                        tbl_hbm.at[pl.ds(lb, 8), :],
                        slots_v.at[g * L + kk],
                        sem,
                    )
            for _ in range(chunk):
                pltpu.make_async_copy(
                    tbl_hbm.at[pl.ds(0, 8), :], slots_v.at[0], sem
                ).wait()
            for g in range(groups_per_chunk):
                lab16 = lab_v[pl.ds(c * chunk + g * L, L)]
                for kk in range(L):
                    i = g * L + kk
                    h = lab16[kk] & 7
                    for dd in range(D // L):
                        outb_v[c * chunk + i, pl.ds(dd * L, L)] = (
                            slots_v[i, h, pl.ds(dd * L, L)]
                        )
            return carry

        lax.fori_loop(0, n_chunks, per_chunk, 0)
        pltpu.sync_copy(outb_v, out_hbm.at[pl.ds(base, b_per_w)])

    return k


def kernel(labels, table):
    B, = labels.shape
    V, D = table.shape
    out128 = _make_gather(V, D, B)(table, labels.astype(jnp.int32))
    return out128[:, :D]


# speedup vs baseline: 1.8046x; 1.0183x over previous
import functools

import jax
import jax.numpy as jnp
from jax import lax
from jax.experimental import pallas as pl
from jax.experimental.pallas import tpu as pltpu
from jax.experimental.pallas import tpu_sc as plsc


@functools.lru_cache(maxsize=None)
def _make_gather(V, D, B):
    info = plsc.get_sparse_core_info()
    NC, NS, L = info.num_cores, info.num_subcores, info.num_lanes
    NW = NC * NS
    assert B % (8 * NW) == 0 and D == 64
    b_per_w = B // NW
    chunk = 16
    n_chunks = b_per_w // chunk
    groups_per_chunk = chunk // L
    mesh = plsc.VectorSubcoreMesh(core_axis_name="c", subcore_axis_name="s")

    @functools.partial(
        pl.kernel,
        mesh=mesh,
        out_type=jax.ShapeDtypeStruct((B, 2 * D), jnp.float32),
        scratch_types=[
            pltpu.VMEM((b_per_w,), jnp.int32),
            pltpu.VMEM((2, chunk, 8, D), jnp.float32),
            pltpu.VMEM((b_per_w, 2 * D), jnp.float32),
            pltpu.SemaphoreType.DMA((2,)),
        ],
    )
    def k(tbl_hbm, idx_hbm, out_hbm, lab_v, slots_v, outb_v, sem):
        wid = lax.axis_index("s") * NC + lax.axis_index("c")
        base = wid * b_per_w
        pltpu.sync_copy(idx_hbm.at[pl.ds(base, b_per_w)], lab_v)

        def fire(c):
            buf = c % 2
            for g in range(groups_per_chunk):
                lab16 = lab_v[pl.ds(c * chunk + g * L, L)]
                for kk in range(L):
                    lb = pl.multiple_of((lab16[kk] >> 3) * 8, 8)
                    pltpu.async_copy(
                        tbl_hbm.at[pl.ds(lb, 8), :],
                        slots_v.at[buf, g * L + kk],
                        sem.at[buf],
                    )

        fire(0)

        def per_chunk(c, carry):
            buf = c % 2

            @pl.when(c + 1 < n_chunks)
            def _():
                fire(c + 1)

            for _ in range(chunk):
                pltpu.make_async_copy(
                    tbl_hbm.at[pl.ds(0, 8), :], slots_v.at[0, 0], sem.at[buf]
                ).wait()
            for g in range(groups_per_chunk):
                lab16 = lab_v[pl.ds(c * chunk + g * L, L)]
                for kk in range(L):
                    i = g * L + kk
                    h = lab16[kk] & 7
                    for dd in range(D // L):
                        outb_v[c * chunk + i, pl.ds(dd * L, L)] = (
                            slots_v[buf, i, h, pl.ds(dd * L, L)]
                        )
            return carry

        lax.fori_loop(0, n_chunks, per_chunk, 0)
        pltpu.sync_copy(outb_v, out_hbm.at[pl.ds(base, b_per_w)])

    return k


def kernel(labels, table):
    B, = labels.shape
    V, D = table.shape
    out128 = _make_gather(V, D, B)(table, labels.astype(jnp.int32))
    return out128[:, :D]
